# baseline (speedup 1.0000x reference)
import functools

import jax
import jax.numpy as jnp
from jax import lax
from jax.experimental import pallas as pl
from jax.experimental.pallas import tpu as pltpu
from jax.experimental.pallas import tpu_sc as plsc

D_MODEL = 768
D_FF = 768
E = 64
K = 2
T = 2048
TILE = 128
NT = 96
R_MAX = NT * TILE
NT_PAD = 128
NC, NS = 2, 16
NW = NC * NS
CHUNK = (T * K) // NW
TOK = T // NW
LANES = 16


def _routing_body(x_ref, gw_ref, gb_ref, r_ref, w0_ref, w1_ref, eot_ref,
                  tix_ref, mf_ref, pos_ref):
    xv = x_ref[...]
    logits = jnp.dot(xv, gw_ref[...], preferred_element_type=jnp.float32)
    logits = logits + gb_ref[...]
    eidx = lax.broadcasted_iota(jnp.int32, (T, E), 1)

    m1 = jnp.max(logits, axis=1, keepdims=True)
    a1 = jnp.min(jnp.where(logits == m1, eidx, E), axis=1, keepdims=True)
    masked = jnp.where(eidx == a1, -jnp.inf, logits)
    m2 = jnp.max(masked, axis=1, keepdims=True)
    a2 = jnp.min(jnp.where(masked == m2, eidx, E), axis=1, keepdims=True)

    w0 = 1.0 / (1.0 + jnp.exp(m2 - m1))
    w1 = 1.0 - w0

    mf = ((eidx == a1) | (eidx == a2)).astype(jnp.float32)
    mf_ref[...] = mf

    R = 256
    ti = lax.broadcasted_iota(jnp.int32, (R, R), 0)
    tj = lax.broadcasted_iota(jnp.int32, (R, R), 1)
    tril = (tj < ti).astype(jnp.float32)

    def body(i, carry):
        blk = mf_ref[pl.ds(i * R, R), :]
        pos_ref[pl.ds(i * R, R), :] = (
            jnp.dot(tril, blk, preferred_element_type=jnp.float32) + carry)
        return carry + jnp.sum(blk, axis=0, keepdims=True)

    counts = lax.fori_loop(0, T // R, body, jnp.zeros((1, E), jnp.float32))

    pc = jnp.ceil(counts / float(TILE)) * float(TILE)
    ei = lax.broadcasted_iota(jnp.int32, (E, E), 0)
    ej = lax.broadcasted_iota(jnp.int32, (E, E), 1)
    triu = (ei < ej).astype(jnp.float32)
    off = jnp.dot(pc, triu, preferred_element_type=jnp.float32)

    tgt = off + pos_ref[...]
    r0 = jnp.sum(jnp.where(eidx == a1, tgt, 0.0), axis=1, keepdims=True)
    r1 = jnp.sum(jnp.where(eidx == a2, tgt, 0.0), axis=1, keepdims=True)
    r_ref[...] = jnp.concatenate(
        [r0.astype(jnp.int32), r1.astype(jnp.int32)], axis=1)

    w0_ref[...] = jnp.broadcast_to(w0, (T, LANES))
    w1_ref[...] = jnp.broadcast_to(w1, (T, LANES))

    ends = off + pc
    eye = (ei == ej).astype(jnp.float32)
    ends_col = jnp.sum(jnp.broadcast_to(ends, (E, E)) * eye, axis=1,
                       keepdims=True)
    nact = jnp.sum(pc, axis=1, keepdims=True) / float(TILE)
    tstart = (lax.broadcasted_iota(jnp.int32, (E, NT_PAD), 1)
              .astype(jnp.float32) * float(TILE))
    tstart = jnp.minimum(tstart, (nact - 1.0) * float(TILE))
    cmp = (ends_col <= tstart).astype(jnp.int32)
    eot = jnp.sum(cmp, axis=0, keepdims=True)
    eot_ref[...] = jnp.minimum(eot, E - 1)
    ti = lax.broadcasted_iota(jnp.int32, (1, NT_PAD), 1)
    tix_ref[...] = jnp.minimum(ti, nact.astype(jnp.int32) - 1)


_routing_call = pl.pallas_call(
    _routing_body,
    out_shape=[
        jax.ShapeDtypeStruct((T, K), jnp.int32),
        jax.ShapeDtypeStruct((T, LANES), jnp.float32),
        jax.ShapeDtypeStruct((T, LANES), jnp.float32),
        jax.ShapeDtypeStruct((1, NT_PAD), jnp.int32),
        jax.ShapeDtypeStruct((1, NT_PAD), jnp.int32),
    ],
    scratch_shapes=[
        pltpu.VMEM((T, E), jnp.float32),
        pltpu.VMEM((T, E), jnp.float32),
    ],
)


def _ffn_body(eot_ref, tix_ref, tok_ref, x_ref, w1_ref, b1_ref, w2_ref,
              b2_ref, y_ref, xg_ref):
    del eot_ref
    i = pl.program_id(0)

    @pl.when(i <= tix_ref[NT_PAD - 1])
    def _():
        def gather(j, carry):
            t = tok_ref[i * TILE + j]
            t = jnp.minimum(jnp.maximum(t, 0), T - 1)
            xg_ref[pl.ds(j, 1), :] = x_ref[pl.ds(t, 1), :]
            return carry

        lax.fori_loop(0, TILE, gather, 0)
        h = jnp.dot(xg_ref[...], w1_ref[0], preferred_element_type=jnp.float32)
        h = jnp.maximum(h + b1_ref[0], 0.0)
        y = jnp.dot(h, w2_ref[0], preferred_element_type=jnp.float32)
        y_ref[...] = y + b2_ref[0]


_ffn_call = pl.pallas_call(
    _ffn_body,
    grid_spec=pltpu.PrefetchScalarGridSpec(
        num_scalar_prefetch=3,
        grid=(NT,),
        in_specs=[
            pl.BlockSpec((T, D_MODEL), lambda i, eot, tix, tok: (0, 0)),
            pl.BlockSpec((1, D_MODEL, D_FF),
                         lambda i, eot, tix, tok: (eot[i], 0, 0)),
            pl.BlockSpec((1, 1, D_FF), lambda i, eot, tix, tok: (eot[i], 0, 0)),
            pl.BlockSpec((1, D_FF, D_MODEL),
                         lambda i, eot, tix, tok: (eot[i], 0, 0)),
            pl.BlockSpec((1, 1, D_MODEL),
                         lambda i, eot, tix, tok: (eot[i], 0, 0)),
        ],
        out_specs=pl.BlockSpec((TILE, D_MODEL),
                               lambda i, eot, tix, tok: (tix[i], 0)),
        scratch_shapes=[pltpu.VMEM((TILE, D_MODEL), jnp.float32)],
    ),
    out_shape=jax.ShapeDtypeStruct((R_MAX, D_MODEL), jnp.float32),
)


@functools.lru_cache(maxsize=1)
def _sc_kernels():
    mesh = plsc.VectorSubcoreMesh(
        core_axis_name="c", subcore_axis_name="s",
        num_cores=NC, num_subcores=NS)

    @functools.partial(
        pl.kernel,
        out_type=jax.ShapeDtypeStruct((R_MAX, 128), jnp.int32),
        mesh=mesh,
        scratch_types=[
            pltpu.VMEM((CHUNK,), jnp.int32),
            pltpu.VMEM((CHUNK, 128), jnp.int32),
            pltpu.SemaphoreType.DMA,
        ],
    )
    def dispatch(tokc_hbm, ridx_hbm, tof_hbm, idx_v, tbuf, sem):
        wid = lax.axis_index("s") * NC + lax.axis_index("c")
        base = pl.multiple_of(wid * CHUNK, CHUNK)
        pltpu.sync_copy(ridx_hbm.at[pl.ds(base, CHUNK)], idx_v)
        tok = pl.multiple_of(jnp.bitwise_and(base, T - 1), CHUNK)
        pltpu.sync_copy(tokc_hbm.at[pl.ds(tok, CHUNK)], tbuf)
        pltpu.async_copy(tbuf, tof_hbm.at[idx_v], sem).wait()

    @functools.partial(
        pl.kernel,
        out_type=jax.ShapeDtypeStruct((T, D_MODEL), jnp.float32),
        mesh=mesh,
        scratch_types=[
            pltpu.VMEM((TOK,), jnp.int32),
            pltpu.VMEM((TOK,), jnp.int32),
            pltpu.VMEM((TOK, D_MODEL), jnp.float32),
            pltpu.VMEM((TOK, D_MODEL), jnp.float32),
            pltpu.VMEM((TOK, LANES), jnp.float32),
            pltpu.VMEM((TOK, LANES), jnp.float32),
            pltpu.SemaphoreType.DMA,
            pltpu.SemaphoreType.DMA,
        ],
    )
    def combine(y_hbm, ridx_hbm, w0_hbm, w1_hbm, out_hbm,
                i0, i1, b0, b1, wb0, wb1, s0, s1):
        wid = lax.axis_index("s") * NC + lax.axis_index("c")
        base = pl.multiple_of(wid * TOK, TOK)
        pltpu.sync_copy(ridx_hbm.at[pl.ds(base, TOK)], i0)
        pltpu.sync_copy(ridx_hbm.at[pl.ds(T + base, TOK)], i1)
        pltpu.sync_copy(w0_hbm.at[pl.ds(base, TOK)], wb0)
        pltpu.sync_copy(w1_hbm.at[pl.ds(base, TOK)], wb1)
        c0 = pltpu.async_copy(y_hbm.at[i0], b0, s0)
        c1 = pltpu.async_copy(y_hbm.at[i1], b1, s1)
        c0.wait()
        c1.wait()

        def row(j, carry):
            wv0 = wb0[j, pl.ds(0, LANES)]
            wv1 = wb1[j, pl.ds(0, LANES)]
            for c in range(D_MODEL // LANES):
                sl = pl.ds(c * LANES, LANES)
                b0[j, sl] = wv0 * b0[j, sl] + wv1 * b1[j, sl]
            return carry

        lax.fori_loop(0, TOK, row, 0)
        pltpu.sync_copy(b0, out_hbm.at[pl.ds(base, TOK)])

    return dispatch, combine


@jax.jit
def _moe(x, gate_W, gate_b, W1, b1, W2, b2):
    x2d = x.reshape(T, D_MODEL)
    dispatch, combine = _sc_kernels()
    r, w0b, w1b, eot, tix = _routing_call(x2d, gate_W, gate_b.reshape(1, E))
    ridx = jnp.concatenate([r[:, 0], r[:, 1]], axis=0)
    tokc = jnp.broadcast_to(
        jnp.arange(T, dtype=jnp.int32)[:, None], (T, 128))
    tok_of_row = dispatch(tokc, ridx)
    y_sorted = _ffn_call(eot.reshape(NT_PAD), tix.reshape(NT_PAD),
                         tok_of_row[:, 0], x2d,
                         W1, b1.reshape(E, 1, D_FF),
                         W2, b2.reshape(E, 1, D_MODEL))
    out = combine(y_sorted, ridx, w0b, w1b)
    return out.reshape(x.shape)


def kernel(x, gate_W, gate_b, W1, b1, W2, b2):
    return _moe(x, gate_W, gate_b, W1, b1, W2, b2)


# speedup vs baseline: 1.3083x; 1.3083x over previous
import functools

import jax
import jax.numpy as jnp
from jax import lax
from jax.experimental import pallas as pl
from jax.experimental.pallas import tpu as pltpu
from jax.experimental.pallas import tpu_sc as plsc

D_MODEL = 768
D_FF = 768
E = 64
K = 2
T = 2048
TILE = 128
NT = 96
NT2 = NT // 2
R_MAX = NT * TILE
NT_PAD = 128
NC, NS = 2, 16
NW = NC * NS
CHUNK = (T * K) // NW
TOK = T // NW
LANES = 16


def _routing_body(x_ref, gw_ref, gb_ref, r_ref, w0_ref, w1_ref, eot_ref,
                  tix2_ref, mf_ref, pos_ref):
    xv = x_ref[...]
    logits = jnp.dot(xv, gw_ref[...], preferred_element_type=jnp.float32)
    logits = logits + gb_ref[...]
    eidx = lax.broadcasted_iota(jnp.int32, (T, E), 1)

    m1 = jnp.max(logits, axis=1, keepdims=True)
    a1 = jnp.min(jnp.where(logits == m1, eidx, E), axis=1, keepdims=True)
    masked = jnp.where(eidx == a1, -jnp.inf, logits)
    m2 = jnp.max(masked, axis=1, keepdims=True)
    a2 = jnp.min(jnp.where(masked == m2, eidx, E), axis=1, keepdims=True)

    w0 = 1.0 / (1.0 + jnp.exp(m2 - m1))
    w1 = 1.0 - w0

    mf = ((eidx == a1) | (eidx == a2)).astype(jnp.float32)
    mf_ref[...] = mf

    R = 256
    ti = lax.broadcasted_iota(jnp.int32, (R, R), 0)
    tj = lax.broadcasted_iota(jnp.int32, (R, R), 1)
    tril = (tj < ti).astype(jnp.float32)

    def body(i, carry):
        blk = mf_ref[pl.ds(i * R, R), :]
        pos_ref[pl.ds(i * R, R), :] = (
            jnp.dot(tril, blk, preferred_element_type=jnp.float32) + carry)
        return carry + jnp.sum(blk, axis=0, keepdims=True)

    counts = lax.fori_loop(0, T // R, body, jnp.zeros((1, E), jnp.float32))

    pc = jnp.ceil(counts / float(TILE)) * float(TILE)
    ei = lax.broadcasted_iota(jnp.int32, (E, E), 0)
    ej = lax.broadcasted_iota(jnp.int32, (E, E), 1)
    triu = (ei < ej).astype(jnp.float32)
    off = jnp.dot(pc, triu, preferred_element_type=jnp.float32)

    tgt = off + pos_ref[...]
    r0 = jnp.sum(jnp.where(eidx == a1, tgt, 0.0), axis=1, keepdims=True)
    r1 = jnp.sum(jnp.where(eidx == a2, tgt, 0.0), axis=1, keepdims=True)
    r_ref[...] = jnp.concatenate(
        [r0.astype(jnp.int32), r1.astype(jnp.int32)], axis=1)

    w0_ref[...] = jnp.broadcast_to(w0, (T, LANES))
    w1_ref[...] = jnp.broadcast_to(w1, (T, LANES))

    ends = off + pc
    eye = (ei == ej).astype(jnp.float32)
    ends_col = jnp.sum(jnp.broadcast_to(ends, (E, E)) * eye, axis=1,
                       keepdims=True)
    nact = jnp.sum(pc, axis=1, keepdims=True) / float(TILE)
    tstart = (lax.broadcasted_iota(jnp.int32, (E, NT_PAD), 1)
              .astype(jnp.float32) * float(TILE))
    tstart = jnp.minimum(tstart, (nact - 1.0) * float(TILE))
    cmp = (ends_col <= tstart).astype(jnp.int32)
    eot = jnp.sum(cmp, axis=0, keepdims=True)
    eot_ref[...] = jnp.minimum(eot, E - 1)
    nact2 = jnp.ceil(nact / 2.0).astype(jnp.int32)
    ti = lax.broadcasted_iota(jnp.int32, (1, NT_PAD), 1)
    tix2_ref[...] = jnp.minimum(ti, nact2 - 1)


_routing_call = pl.pallas_call(
    _routing_body,
    out_shape=[
        jax.ShapeDtypeStruct((T, K), jnp.int32),
        jax.ShapeDtypeStruct((T, LANES), jnp.float32),
        jax.ShapeDtypeStruct((T, LANES), jnp.float32),
        jax.ShapeDtypeStruct((1, NT_PAD), jnp.int32),
        jax.ShapeDtypeStruct((1, NT_PAD), jnp.int32),
    ],
    scratch_shapes=[
        pltpu.VMEM((T, E), jnp.float32),
        pltpu.VMEM((T, E), jnp.float32),
    ],
)


def _ffn_body(eot_ref, tix2_ref, x_ref, w1a_ref, b1a_ref, w2a_ref, b2a_ref,
              w1b_ref, b1b_ref, w2b_ref, b2b_ref, y_ref):
    del eot_ref

    @pl.when(pl.program_id(0) <= tix2_ref[NT_PAD - 1])
    def _():
        xa = x_ref[:TILE]
        ha = jnp.dot(xa, w1a_ref[0], preferred_element_type=jnp.float32)
        ha = jnp.maximum(ha + b1a_ref[0], 0.0)
        ya = jnp.dot(ha, w2a_ref[0], preferred_element_type=jnp.float32)
        y_ref[:TILE] = ya + b2a_ref[0]
        xb = x_ref[TILE:]
        hb = jnp.dot(xb, w1b_ref[0], preferred_element_type=jnp.float32)
        hb = jnp.maximum(hb + b1b_ref[0], 0.0)
        yb = jnp.dot(hb, w2b_ref[0], preferred_element_type=jnp.float32)
        y_ref[TILE:] = yb + b2b_ref[0]


_ffn_call = pl.pallas_call(
    _ffn_body,
    grid_spec=pltpu.PrefetchScalarGridSpec(
        num_scalar_prefetch=2,
        grid=(NT2,),
        in_specs=[
            pl.BlockSpec((2 * TILE, D_MODEL),
                         lambda i, eot, tix2: (tix2[i], 0)),
            pl.BlockSpec((1, D_MODEL, D_FF),
                         lambda i, eot, tix2: (eot[2 * i], 0, 0)),
            pl.BlockSpec((1, 1, D_FF),
                         lambda i, eot, tix2: (eot[2 * i], 0, 0)),
            pl.BlockSpec((1, D_FF, D_MODEL),
                         lambda i, eot, tix2: (eot[2 * i], 0, 0)),
            pl.BlockSpec((1, 1, D_MODEL),
                         lambda i, eot, tix2: (eot[2 * i], 0, 0)),
            pl.BlockSpec((1, D_MODEL, D_FF),
                         lambda i, eot, tix2: (eot[2 * i + 1], 0, 0)),
            pl.BlockSpec((1, 1, D_FF),
                         lambda i, eot, tix2: (eot[2 * i + 1], 0, 0)),
            pl.BlockSpec((1, D_FF, D_MODEL),
                         lambda i, eot, tix2: (eot[2 * i + 1], 0, 0)),
            pl.BlockSpec((1, 1, D_MODEL),
                         lambda i, eot, tix2: (eot[2 * i + 1], 0, 0)),
        ],
        out_specs=pl.BlockSpec((2 * TILE, D_MODEL),
                               lambda i, eot, tix2: (tix2[i], 0)),
    ),
    out_shape=jax.ShapeDtypeStruct((R_MAX, D_MODEL), jnp.float32),
)


@functools.lru_cache(maxsize=1)
def _sc_kernels():
    mesh = plsc.VectorSubcoreMesh(
        core_axis_name="c", subcore_axis_name="s",
        num_cores=NC, num_subcores=NS)

    @functools.partial(
        pl.kernel,
        out_type=jax.ShapeDtypeStruct((R_MAX, D_MODEL), jnp.float32),
        mesh=mesh,
        scratch_types=[
            pltpu.VMEM((CHUNK,), jnp.int32),
            pltpu.VMEM((CHUNK, D_MODEL), jnp.float32),
            pltpu.SemaphoreType.DMA,
        ],
    )
    def dispatch(x_hbm, ridx_hbm, xs_hbm, idx_v, xbuf, sem):
        wid = lax.axis_index("s") * NC + lax.axis_index("c")
        base = pl.multiple_of(wid * CHUNK, CHUNK)
        pltpu.sync_copy(ridx_hbm.at[pl.ds(base, CHUNK)], idx_v)
        tok = pl.multiple_of(jnp.bitwise_and(base, T - 1), CHUNK)
        pltpu.sync_copy(x_hbm.at[pl.ds(tok, CHUNK)], xbuf)
        pltpu.async_copy(xbuf, xs_hbm.at[idx_v], sem).wait()

    @functools.partial(
        pl.kernel,
        out_type=jax.ShapeDtypeStruct((T, D_MODEL), jnp.float32),
        mesh=mesh,
        scratch_types=[
            pltpu.VMEM((TOK,), jnp.int32),
            pltpu.VMEM((TOK,), jnp.int32),
            pltpu.VMEM((TOK, D_MODEL), jnp.float32),
            pltpu.VMEM((TOK, D_MODEL), jnp.float32),
            pltpu.VMEM((TOK, LANES), jnp.float32),
            pltpu.VMEM((TOK, LANES), jnp.float32),
            pltpu.SemaphoreType.DMA,
            pltpu.SemaphoreType.DMA,
        ],
    )
    def combine(y_hbm, ridx_hbm, w0_hbm, w1_hbm, out_hbm,
                i0, i1, b0, b1, wb0, wb1, s0, s1):
        wid = lax.axis_index("s") * NC + lax.axis_index("c")
        base = pl.multiple_of(wid * TOK, TOK)
        pltpu.sync_copy(ridx_hbm.at[pl.ds(base, TOK)], i0)
        pltpu.sync_copy(ridx_hbm.at[pl.ds(T + base, TOK)], i1)
        pltpu.sync_copy(w0_hbm.at[pl.ds(base, TOK)], wb0)
        pltpu.sync_copy(w1_hbm.at[pl.ds(base, TOK)], wb1)
        c0 = pltpu.async_copy(y_hbm.at[i0], b0, s0)
        c1 = pltpu.async_copy(y_hbm.at[i1], b1, s1)
        c0.wait()
        c1.wait()

        def row(j, carry):
            wv0 = wb0[j, pl.ds(0, LANES)]
            wv1 = wb1[j, pl.ds(0, LANES)]
            for c in range(D_MODEL // LANES):
                sl = pl.ds(c * LANES, LANES)
                b0[j, sl] = wv0 * b0[j, sl] + wv1 * b1[j, sl]
            return carry

        lax.fori_loop(0, TOK, row, 0)
        pltpu.sync_copy(b0, out_hbm.at[pl.ds(base, TOK)])

    return dispatch, combine


@jax.jit
def _moe(x, gate_W, gate_b, W1, b1, W2, b2):
    x2d = x.reshape(T, D_MODEL)
    dispatch, combine = _sc_kernels()
    r, w0b, w1b, eot, tix2 = _routing_call(x2d, gate_W, gate_b.reshape(1, E))
    ridx = jnp.concatenate([r[:, 0], r[:, 1]], axis=0)
    x_sorted = dispatch(x2d, ridx)
    b1r = b1.reshape(E, 1, D_FF)
    b2r = b2.reshape(E, 1, D_MODEL)
    y_sorted = _ffn_call(eot.reshape(NT_PAD), tix2.reshape(NT_PAD), x_sorted,
                         W1, b1r, W2, b2r, W1, b1r, W2, b2r)
    out = combine(y_sorted, ridx, w0b, w1b)
    return out.reshape(x.shape)


def kernel(x, gate_W, gate_b, W1, b1, W2, b2):
    return _moe(x, gate_W, gate_b, W1, b1, W2, b2)
